# Initial kernel scaffold; baseline (speedup 1.0000x reference)
import jax
import jax.numpy as jnp
from jax.experimental import pallas as pl


def kernel(results, indexes, features, labels):
    raise NotImplementedError("write your pallas kernel here")


# speedup vs baseline: 19.9190x; 19.9190x over previous
import functools

import jax
import jax.numpy as jnp
from jax import lax
from jax.experimental import pallas as pl
from jax.experimental.pallas import tpu as pltpu
from jax.experimental.pallas import tpu_sc as plsc

M = 100000
F = 128
B = 4096
C = 1000
TEMP = 0.05

NC = 2
NS = 16
NW = NC * NS

CHUNK = 128
M_PAD = 102400
ROWS_PER_W = M_PAD // NW
CHUNKS_PER_W = ROWS_PER_W // CHUNK
C_PAD = 1024
B_PER_W = B // NW

@functools.cache
def _build_sc_segsum():
    mesh = plsc.VectorSubcoreMesh(core_axis_name="c", subcore_axis_name="s")

    @functools.partial(
        pl.kernel,
        mesh=mesh,
        out_type=(
            jax.ShapeDtypeStruct((NC, C_PAD, F), jnp.float32),
            jax.ShapeDtypeStruct((NC, C_PAD, 16), jnp.float32),
            jax.ShapeDtypeStruct((B,), jnp.int32),
        ),
        scratch_types=[
            pltpu.VMEM((CHUNK, F), jnp.float32),
            pltpu.VMEM((CHUNKS_PER_W, CHUNK), jnp.int32),
            pltpu.VMEM((CHUNK, 16), jnp.float32),
            pltpu.VMEM((B_PER_W,), jnp.int32),
            pltpu.VMEM((B_PER_W,), jnp.int32),
            pltpu.VMEM_SHARED((C_PAD, F), jnp.float32),
            pltpu.VMEM_SHARED((C_PAD, 16), jnp.float32),
        ],
    )
    def sc_segsum(feat_hbm, lbl2d_hbm, lbl1d_hbm, idx_hbm, ones_hbm, zg_hbm,
                  zn_hbm, partials_hbm, counts_hbm, targets_hbm,
                  feat_v, lbl_v, ones_v, idx_v, tgt_v, acc_g, acc_n):
        _sc_body(feat_hbm, lbl2d_hbm, lbl1d_hbm, idx_hbm, ones_hbm, zg_hbm,
                 zn_hbm, partials_hbm, counts_hbm, targets_hbm,
                 feat_v, lbl_v, ones_v, idx_v, tgt_v, acc_g, acc_n)

    return sc_segsum


def _sc_body(feat_hbm, lbl2d_hbm, lbl1d_hbm, idx_hbm, ones_hbm, zg_hbm,
             zn_hbm, partials_hbm, counts_hbm, targets_hbm,
             feat_v, lbl_v, ones_v, idx_v, tgt_v, acc_g, acc_n):
    c = lax.axis_index("c")
    s = lax.axis_index("s")
    w = s * NC + c

    @pl.when(s == 0)
    def _():
        pltpu.sync_copy(zg_hbm, acc_g)
        pltpu.sync_copy(zn_hbm, acc_n)

    pltpu.sync_copy(ones_hbm, ones_v)
    pltpu.sync_copy(lbl2d_hbm.at[w], lbl_v)
    plsc.subcore_barrier()

    def body(j, carry):
        off = (w * CHUNKS_PER_W + j) * CHUNK
        pltpu.sync_copy(feat_hbm.at[pl.ds(off, CHUNK)], feat_v)
        pltpu.sync_copy(feat_v, acc_g.at[lbl_v.at[j]], add=True)
        pltpu.sync_copy(ones_v, acc_n.at[lbl_v.at[j]], add=True)
        return carry

    lax.fori_loop(0, CHUNKS_PER_W, body, 0)

    pltpu.sync_copy(idx_hbm.at[pl.ds(w * B_PER_W, B_PER_W)], idx_v)
    pltpu.sync_copy(lbl1d_hbm.at[idx_v], tgt_v)
    pltpu.sync_copy(tgt_v, targets_hbm.at[pl.ds(w * B_PER_W, B_PER_W)])

    plsc.subcore_barrier()

    @pl.when(s == 0)
    def _():
        pltpu.sync_copy(acc_g, partials_hbm.at[c])
        pltpu.sync_copy(acc_n, counts_hbm.at[c])


_BLK = 512
_GRID = B // _BLK


def _tc_body(res_ref, part_ref, nums_ref, tgt_ref, out_ref, acc):
    i = pl.program_id(0)

    x = res_ref[...]
    norm = jnp.sqrt(jnp.sum(x * x, axis=1, keepdims=True))
    x = x / jnp.maximum(norm, 1e-12)

    g = part_ref[0] + part_ref[1]
    vec = lax.dot_general(x, g, (((1,), (1,)), ((), ())),
                          preferred_element_type=jnp.float32,
                          precision=lax.Precision.HIGHEST)

    nums = nums_ref[...]
    col = lax.broadcasted_iota(jnp.int32, (1, C_PAD), 1)
    mask = jnp.where((nums > 0.0) & (col < C), 1.0, 0.0)
    scale = 1.0 / (TEMP * jnp.maximum(nums, 1.0))
    vec = vec * scale

    exps = jnp.exp(vec) * mask
    sums = jnp.sum(exps, axis=1, keepdims=True) + 1e-6
    logp = jnp.log(exps / sums + 1e-6)

    t = tgt_ref[...]
    hit = lax.broadcasted_iota(jnp.int32, (_BLK, C_PAD), 1) == t
    block_sum = jnp.sum(jnp.where(hit, logp, 0.0))

    @pl.when(i == 0)
    def _():
        acc[0] = 0.0

    acc[0] += block_sum

    @pl.when(i == _GRID - 1)
    def _():
        out_ref[...] = jnp.full((1, 1), -acc[0] / float(B), jnp.float32)


def _tc_loss(results, partials, nums_row, targets_f):
    return pl.pallas_call(
        _tc_body,
        grid=(_GRID,),
        in_specs=[
            pl.BlockSpec((_BLK, F), lambda i: (i, 0)),
            pl.BlockSpec((NC, C_PAD, F), lambda i: (0, 0, 0)),
            pl.BlockSpec((1, C_PAD), lambda i: (0, 0)),
            pl.BlockSpec((_BLK, 1), lambda i: (i, 0)),
        ],
        out_specs=pl.BlockSpec((1, 1), lambda i: (0, 0)),
        out_shape=jax.ShapeDtypeStruct((1, 1), jnp.float32),
        scratch_shapes=[pltpu.SMEM((1,), jnp.float32)],
    )(results, partials, nums_row, targets_f)


def kernel(results, indexes, features, labels):
    feat_pad = jnp.concatenate(
        [features, jnp.zeros((M_PAD - M, F), jnp.float32)], axis=0)
    lbl_pad = jnp.concatenate(
        [labels, jnp.full((M_PAD - M,), C, jnp.int32)], axis=0)
    lbl2d = lbl_pad.reshape(NW, CHUNKS_PER_W, CHUNK)
    ones_rows = jnp.ones((CHUNK, 16), jnp.float32)
    zg = jnp.zeros((C_PAD, F), jnp.float32)
    zn = jnp.zeros((C_PAD, 16), jnp.float32)

    partials, counts, targets = _build_sc_segsum()(
        feat_pad, lbl2d, lbl_pad, indexes, ones_rows, zg, zn)

    nums_row = (counts[0, :, 0] + counts[1, :, 0]).reshape(1, C_PAD)
    targets_f = targets.reshape(B, 1)

    loss = _tc_loss(results, partials, nums_row, targets_f)
    return loss.reshape(())


# speedup vs baseline: 30.6269x; 1.5376x over previous
import functools

import jax
import jax.numpy as jnp
from jax import lax
from jax.experimental import pallas as pl
from jax.experimental.pallas import tpu as pltpu
from jax.experimental.pallas import tpu_sc as plsc

M = 100000
F = 128
B = 4096
C = 1000
TEMP = 0.05

NC = 2
NS = 16
NW = NC * NS

CHUNK = 128
NFULL = M // CHUNK
TAIL = M - NFULL * CHUNK
MAXQ = -(-NFULL // NW)
NQ_EXTRA = NFULL - (MAXQ - 1) * NW
C_PAD = 1024
B_PER_W = B // NW


@functools.cache
def _build_sc_segsum():
    mesh = plsc.VectorSubcoreMesh(core_axis_name="c", subcore_axis_name="s")

    @functools.partial(
        pl.kernel,
        mesh=mesh,
        out_type=(
            jax.ShapeDtypeStruct((NC, C_PAD, F), jnp.float32),
            jax.ShapeDtypeStruct((NC, C_PAD, 16), jnp.float32),
            jax.ShapeDtypeStruct((B,), jnp.int32),
        ),
        scratch_types=[
            pltpu.VMEM((2, CHUNK, F), jnp.float32),
            pltpu.VMEM((2, CHUNK), jnp.int32),
            pltpu.VMEM((TAIL,), jnp.int32),
            pltpu.VMEM((CHUNK, 16), jnp.float32),
            pltpu.VMEM((B_PER_W,), jnp.int32),
            pltpu.VMEM((B_PER_W,), jnp.int32),
            pltpu.VMEM_SHARED((C_PAD, F), jnp.float32),
            pltpu.VMEM_SHARED((C_PAD, 16), jnp.float32),
            pltpu.SemaphoreType.DMA((2,)),
            pltpu.SemaphoreType.DMA((2,)),
            pltpu.SemaphoreType.DMA,
            pltpu.SemaphoreType.DMA,
        ],
    )
    def sc_segsum(feat_hbm, lbl_hbm, idx_hbm, ones_hbm, zg_hbm, zn_hbm,
                  partials_hbm, counts_hbm, targets_hbm,
                  feat_v, lbl_v, ltail_v, ones_v, idx_v, tgt_v, acc_g, acc_n,
                  sem_f, sem_l, sem_s1, sem_s2):
        c = lax.axis_index("c")
        s = lax.axis_index("s")
        w = s * NC + c
        nq = jnp.where(w < NQ_EXTRA, MAXQ, MAXQ - 1)

        @pl.when(s == 0)
        def _():
            pltpu.sync_copy(zg_hbm, acc_g)
            pltpu.sync_copy(zn_hbm, acc_n)

        pltpu.sync_copy(ones_hbm, ones_v)
        plsc.subcore_barrier()

        def load(i, slot):
            off = (w + NW * i) * CHUNK
            pltpu.async_copy(
                feat_hbm.at[pl.ds(off, CHUNK)], feat_v.at[slot], sem_f.at[slot])
            pltpu.async_copy(
                lbl_hbm.at[pl.ds(off, CHUNK)], lbl_v.at[slot], sem_l.at[slot])

        def wait_load(i, slot):
            off = (w + NW * i) * CHUNK
            pltpu.make_async_copy(
                feat_hbm.at[pl.ds(off, CHUNK)], feat_v.at[slot],
                sem_f.at[slot]).wait()
            pltpu.make_async_copy(
                lbl_hbm.at[pl.ds(off, CHUNK)], lbl_v.at[slot],
                sem_l.at[slot]).wait()

        load(0, 0)
        for i in range(MAXQ):
            slot = i & 1

            @pl.when(i < nq)
            def _(i=i, slot=slot):
                wait_load(i, slot)
                if i + 1 < MAXQ:
                    @pl.when(i + 1 < nq)
                    def _():
                        load(i + 1, 1 - slot)
                c1 = pltpu.async_copy(
                    feat_v.at[slot], acc_g.at[lbl_v.at[slot]], sem_s1, add=True)
                c2 = pltpu.async_copy(
                    ones_v, acc_n.at[lbl_v.at[slot]], sem_s2, add=True)
                c1.wait()
                c2.wait()

        @pl.when(w == NW - 1)
        def _():
            off = NFULL * CHUNK
            pltpu.sync_copy(feat_hbm.at[pl.ds(off, TAIL)],
                            feat_v.at[0].at[pl.ds(0, TAIL)])
            pltpu.sync_copy(lbl_hbm.at[pl.ds(off, TAIL)], ltail_v)
            pltpu.sync_copy(feat_v.at[0].at[pl.ds(0, TAIL)],
                            acc_g.at[ltail_v], add=True)
            pltpu.sync_copy(ones_v.at[pl.ds(0, TAIL)],
                            acc_n.at[ltail_v], add=True)

        pltpu.sync_copy(idx_hbm.at[pl.ds(w * B_PER_W, B_PER_W)], idx_v)
        pltpu.sync_copy(lbl_hbm.at[idx_v], tgt_v)
        pltpu.sync_copy(tgt_v, targets_hbm.at[pl.ds(w * B_PER_W, B_PER_W)])

        plsc.subcore_barrier()

        @pl.when(s == 0)
        def _():
            pltpu.sync_copy(acc_g, partials_hbm.at[c])
            pltpu.sync_copy(acc_n, counts_hbm.at[c])

    return sc_segsum


_BLK = 512
_GRID = B // _BLK


def _tc_body(res_ref, part_ref, nums_ref, tgt_ref, out_ref, g_ref, acc):
    i = pl.program_id(0)

    @pl.when(i == 0)
    def _():
        g_ref[...] = part_ref[0] + part_ref[1]
        acc[0] = 0.0

    x = res_ref[...]
    norm = jnp.sqrt(jnp.sum(x * x, axis=1, keepdims=True))
    x = x / jnp.maximum(norm, 1e-12)

    vec = lax.dot_general(x, g_ref[...], (((1,), (1,)), ((), ())),
                          preferred_element_type=jnp.float32,
                          precision=lax.Precision.HIGHEST)

    nums = nums_ref[...]
    col = lax.broadcasted_iota(jnp.int32, (1, C_PAD), 1)
    mask = jnp.where((nums > 0.0) & (col < C), 1.0, 0.0)
    scale = 1.0 / (TEMP * jnp.maximum(nums, 1.0))

    mexp = jnp.exp(vec * scale) * mask
    sums = jnp.sum(mexp, axis=1, keepdims=True) + 1e-6

    t = tgt_ref[...]
    hit = lax.broadcasted_iota(jnp.int32, (_BLK, C_PAD), 1) == t
    ex_t = jnp.sum(jnp.where(hit, mexp, 0.0), axis=1, keepdims=True)
    picked = jnp.log(ex_t + 1e-6 * sums) - jnp.log(sums)

    acc[0] += jnp.sum(picked)

    @pl.when(i == _GRID - 1)
    def _():
        out_ref[...] = jnp.full((1, 1), -acc[0] / float(B), jnp.float32)


def _tc_loss(results, partials, nums_row, targets_col):
    return pl.pallas_call(
        _tc_body,
        grid=(_GRID,),
        in_specs=[
            pl.BlockSpec((_BLK, F), lambda i: (i, 0)),
            pl.BlockSpec((NC, C_PAD, F), lambda i: (0, 0, 0)),
            pl.BlockSpec((1, C_PAD), lambda i: (0, 0)),
            pl.BlockSpec((_BLK, 1), lambda i: (i, 0)),
        ],
        out_specs=pl.BlockSpec((1, 1), lambda i: (0, 0)),
        out_shape=jax.ShapeDtypeStruct((1, 1), jnp.float32),
        scratch_shapes=[
            pltpu.VMEM((C_PAD, F), jnp.float32),
            pltpu.SMEM((1,), jnp.float32),
        ],
    )(results, partials, nums_row, targets_col)


def kernel(results, indexes, features, labels):
    ones_rows = jnp.ones((CHUNK, 16), jnp.float32)
    zg = jnp.zeros((C_PAD, F), jnp.float32)
    zn = jnp.zeros((C_PAD, 16), jnp.float32)

    partials, counts, targets = _build_sc_segsum()(
        features, labels, indexes, ones_rows, zg, zn)

    nums_row = (counts[0, :, 0] + counts[1, :, 0]).reshape(1, C_PAD)
    targets_col = targets.reshape(B, 1)

    loss = _tc_loss(results, partials, nums_row, targets_col)
    return loss.reshape(())


# speedup vs baseline: 30.6606x; 1.0011x over previous
import functools

import jax
import jax.numpy as jnp
from jax import lax
from jax.experimental import pallas as pl
from jax.experimental.pallas import tpu as pltpu
from jax.experimental.pallas import tpu_sc as plsc

M = 100000
F = 128
B = 4096
C = 1000
TEMP = 0.05

NC = 2
NS = 16
NW = NC * NS

CHUNK = 128
NFULL = M // CHUNK
TAIL = M - NFULL * CHUNK
MAXQ = -(-NFULL // NW)
NQ_EXTRA = NFULL - (MAXQ - 1) * NW
C_PAD = 1024
B_PER_W = B // NW


@functools.cache
def _build_sc_segsum():
    mesh = plsc.VectorSubcoreMesh(core_axis_name="c", subcore_axis_name="s")

    @functools.partial(
        pl.kernel,
        mesh=mesh,
        out_type=(
            jax.ShapeDtypeStruct((NC, C_PAD, F), jnp.float32),
            jax.ShapeDtypeStruct((NC, C_PAD, 16), jnp.float32),
            jax.ShapeDtypeStruct((B,), jnp.int32),
        ),
        scratch_types=[
            pltpu.VMEM((2, CHUNK, F), jnp.float32),
            pltpu.VMEM((2, CHUNK), jnp.int32),
            pltpu.VMEM((TAIL,), jnp.int32),
            pltpu.VMEM((CHUNK, 16), jnp.float32),
            pltpu.VMEM((B_PER_W,), jnp.int32),
            pltpu.VMEM((B_PER_W,), jnp.int32),
            pltpu.VMEM_SHARED((C_PAD, F), jnp.float32),
            pltpu.VMEM_SHARED((C_PAD, 16), jnp.float32),
            pltpu.SemaphoreType.DMA((2,)),
            pltpu.SemaphoreType.DMA((2,)),
            pltpu.SemaphoreType.DMA((2,)),
            pltpu.SemaphoreType.DMA((2,)),
        ],
    )
    def sc_segsum(feat_hbm, lbl_hbm, idx_hbm, ones_hbm, zg_hbm, zn_hbm,
                  partials_hbm, counts_hbm, targets_hbm,
                  feat_v, lbl_v, ltail_v, ones_v, idx_v, tgt_v, acc_g, acc_n,
                  sem_f, sem_l, sem_s1, sem_s2):
        c = lax.axis_index("c")
        s = lax.axis_index("s")
        w = s * NC + c
        nq = jnp.where(w < NQ_EXTRA, MAXQ, MAXQ - 1)

        @pl.when(s == 0)
        def _():
            pltpu.sync_copy(zg_hbm, acc_g)
            pltpu.sync_copy(zn_hbm, acc_n)

        pltpu.sync_copy(ones_hbm, ones_v)
        plsc.subcore_barrier()

        def load(i, slot):
            off = (w + NW * i) * CHUNK
            pltpu.async_copy(
                feat_hbm.at[pl.ds(off, CHUNK)], feat_v.at[slot], sem_f.at[slot])
            pltpu.async_copy(
                lbl_hbm.at[pl.ds(off, CHUNK)], lbl_v.at[slot], sem_l.at[slot])

        def wait_load(i, slot):
            off = (w + NW * i) * CHUNK
            pltpu.make_async_copy(
                feat_hbm.at[pl.ds(off, CHUNK)], feat_v.at[slot],
                sem_f.at[slot]).wait()
            pltpu.make_async_copy(
                lbl_hbm.at[pl.ds(off, CHUNK)], lbl_v.at[slot],
                sem_l.at[slot]).wait()

        def start_scatter(slot):
            pltpu.async_copy(
                feat_v.at[slot], acc_g.at[lbl_v.at[slot]], sem_s1.at[slot],
                add=True)
            pltpu.async_copy(
                ones_v, acc_n.at[lbl_v.at[slot]], sem_s2.at[slot], add=True)

        def wait_scatter(slot):
            pltpu.make_async_copy(
                feat_v.at[slot], acc_g.at[lbl_v.at[slot]],
                sem_s1.at[slot]).wait()
            pltpu.make_async_copy(
                ones_v, acc_n.at[lbl_v.at[slot]], sem_s2.at[slot]).wait()

        load(0, 0)
        for i in range(MAXQ):
            slot = i & 1

            @pl.when(i < nq)
            def _(i=i, slot=slot):
                wait_load(i, slot)
                if i >= 1:
                    wait_scatter(1 - slot)
                if i + 1 < MAXQ:
                    @pl.when(i + 1 < nq)
                    def _():
                        load(i + 1, 1 - slot)
                start_scatter(slot)

        @pl.when(nq == MAXQ - 1)
        def _():
            wait_scatter((MAXQ - 2) & 1)

        @pl.when(nq == MAXQ)
        def _():
            wait_scatter((MAXQ - 1) & 1)

        @pl.when(w == NW - 1)
        def _():
            off = NFULL * CHUNK
            pltpu.sync_copy(feat_hbm.at[pl.ds(off, TAIL)],
                            feat_v.at[0].at[pl.ds(0, TAIL)])
            pltpu.sync_copy(lbl_hbm.at[pl.ds(off, TAIL)], ltail_v)
            pltpu.sync_copy(feat_v.at[0].at[pl.ds(0, TAIL)],
                            acc_g.at[ltail_v], add=True)
            pltpu.sync_copy(ones_v.at[pl.ds(0, TAIL)],
                            acc_n.at[ltail_v], add=True)

        pltpu.sync_copy(idx_hbm.at[pl.ds(w * B_PER_W, B_PER_W)], idx_v)
        pltpu.sync_copy(lbl_hbm.at[idx_v], tgt_v)
        pltpu.sync_copy(tgt_v, targets_hbm.at[pl.ds(w * B_PER_W, B_PER_W)])

        plsc.subcore_barrier()

        @pl.when(s == 0)
        def _():
            pltpu.sync_copy(acc_g, partials_hbm.at[c])
            pltpu.sync_copy(acc_n, counts_hbm.at[c])

    return sc_segsum


_BLK = 512
_GRID = B // _BLK


def _tc_body(res_ref, part_ref, nums_ref, tgt_ref, out_ref, g_ref, acc):
    i = pl.program_id(0)

    @pl.when(i == 0)
    def _():
        g_ref[...] = part_ref[0] + part_ref[1]
        acc[0] = 0.0

    x = res_ref[...]
    norm = jnp.sqrt(jnp.sum(x * x, axis=1, keepdims=True))
    x = x / jnp.maximum(norm, 1e-12)

    vec = lax.dot_general(x, g_ref[...], (((1,), (1,)), ((), ())),
                          preferred_element_type=jnp.float32,
                          precision=lax.Precision.HIGHEST)

    nums = nums_ref[...]
    col = lax.broadcasted_iota(jnp.int32, (1, C_PAD), 1)
    mask = jnp.where((nums > 0.0) & (col < C), 1.0, 0.0)
    scale = 1.0 / (TEMP * jnp.maximum(nums, 1.0))

    mexp = jnp.exp(vec * scale) * mask
    sums = jnp.sum(mexp, axis=1, keepdims=True) + 1e-6

    t = tgt_ref[...]
    hit = lax.broadcasted_iota(jnp.int32, (_BLK, C_PAD), 1) == t
    ex_t = jnp.sum(jnp.where(hit, mexp, 0.0), axis=1, keepdims=True)
    picked = jnp.log(ex_t + 1e-6 * sums) - jnp.log(sums)

    acc[0] += jnp.sum(picked)

    @pl.when(i == _GRID - 1)
    def _():
        out_ref[...] = jnp.full((1, 1), -acc[0] / float(B), jnp.float32)


def _tc_loss(results, partials, nums_row, targets_col):
    return pl.pallas_call(
        _tc_body,
        grid=(_GRID,),
        in_specs=[
            pl.BlockSpec((_BLK, F), lambda i: (i, 0)),
            pl.BlockSpec((NC, C_PAD, F), lambda i: (0, 0, 0)),
            pl.BlockSpec((1, C_PAD), lambda i: (0, 0)),
            pl.BlockSpec((_BLK, 1), lambda i: (i, 0)),
        ],
        out_specs=pl.BlockSpec((1, 1), lambda i: (0, 0)),
        out_shape=jax.ShapeDtypeStruct((1, 1), jnp.float32),
        scratch_shapes=[
            pltpu.VMEM((C_PAD, F), jnp.float32),
            pltpu.SMEM((1,), jnp.float32),
        ],
    )(results, partials, nums_row, targets_col)


def kernel(results, indexes, features, labels):
    ones_rows = jnp.ones((CHUNK, 16), jnp.float32)
    zg = jnp.zeros((C_PAD, F), jnp.float32)
    zn = jnp.zeros((C_PAD, 16), jnp.float32)

    partials, counts, targets = _build_sc_segsum()(
        features, labels, indexes, ones_rows, zg, zn)

    nums_row = (counts[0, :, 0] + counts[1, :, 0]).reshape(1, C_PAD)
    targets_col = targets.reshape(B, 1)

    loss = _tc_loss(results, partials, nums_row, targets_col)
    return loss.reshape(())
